# baseline (speedup 1.0000x reference)
import jax
import jax.numpy as jnp
from jax import lax
from jax.experimental import pallas as pl
from jax.experimental.pallas import tpu as pltpu

NUM_CLASSES = 1000
C_SPLIT = 896
S_SPLIT = 16
BLOCK_ROWS = 32
NBUF = 8


def _regions(i, out_hbm, sA, sB, sC, sems, slot):
    base = i * BLOCK_ROWS
    return (
        pltpu.make_async_copy(
            sA.at[slot],
            out_hbm.at[pl.ds(base, BLOCK_ROWS), pl.ds(0, S_SPLIT),
                       pl.ds(0, C_SPLIT)],
            sems.at[0, slot]),
        pltpu.make_async_copy(
            sB.at[slot],
            out_hbm.at[pl.ds(base, BLOCK_ROWS), pl.ds(0, S_SPLIT),
                       pl.ds(C_SPLIT, NUM_CLASSES - C_SPLIT)],
            sems.at[1, slot]),
        pltpu.make_async_copy(
            sC.at[slot],
            out_hbm.at[pl.ds(base, BLOCK_ROWS), pl.ds(S_SPLIT, 20 - S_SPLIT),
                       pl.ds(0, NUM_CLASSES)],
            sems.at[2, slot]),
    )


def _onehot_body(x_ref, out_hbm, sA, sB, sC, sems):
    i = pl.program_id(0)
    num = pl.num_programs(0)
    slot = lax.rem(i, NBUF)

    @pl.when(i >= NBUF)
    def _():
        prev = i - NBUF
        for cp in _regions(prev, out_hbm, sA, sB, sC, sems,
                           lax.rem(prev, NBUF)):
            cp.wait()

    idx_ab = x_ref[:, 0:S_SPLIT]
    idx_c = x_ref[:, S_SPLIT:20]
    iota_a = lax.broadcasted_iota(
        jnp.int32, (BLOCK_ROWS, S_SPLIT, C_SPLIT), 2)
    iota_b = lax.broadcasted_iota(
        jnp.int32, (BLOCK_ROWS, S_SPLIT, NUM_CLASSES - C_SPLIT), 2) + C_SPLIT
    iota_c = lax.broadcasted_iota(
        jnp.int32, (BLOCK_ROWS, 20 - S_SPLIT, NUM_CLASSES), 2)
    sA[slot] = (idx_ab[:, :, None] == iota_a).astype(jnp.float32)
    sB[slot] = (idx_ab[:, :, None] == iota_b).astype(jnp.float32)
    sC[slot] = (idx_c[:, :, None] == iota_c).astype(jnp.float32)

    for cp in _regions(i, out_hbm, sA, sB, sC, sems, slot):
        cp.start()

    @pl.when(i == num - 1)
    def _():
        for k in range(NBUF):
            step = num - NBUF + k
            for cp in _regions(step, out_hbm, sA, sB, sC, sems,
                               lax.rem(step, NBUF)):
                cp.wait()


def kernel(x):
    B, S = x.shape
    grid = (B // BLOCK_ROWS,)
    return pl.pallas_call(
        _onehot_body,
        grid=grid,
        in_specs=[pl.BlockSpec((BLOCK_ROWS, S), lambda i: (i, 0))],
        out_specs=pl.BlockSpec(memory_space=pl.ANY),
        out_shape=jax.ShapeDtypeStruct((B, S, NUM_CLASSES), jnp.float32),
        scratch_shapes=[
            pltpu.VMEM((NBUF, BLOCK_ROWS, S_SPLIT, C_SPLIT), jnp.float32),
            pltpu.VMEM((NBUF, BLOCK_ROWS, S_SPLIT, NUM_CLASSES - C_SPLIT),
                       jnp.float32),
            pltpu.VMEM((NBUF, BLOCK_ROWS, 20 - S_SPLIT, NUM_CLASSES),
                       jnp.float32),
            pltpu.SemaphoreType.DMA((3, NBUF)),
        ],
    )(x.astype(jnp.int32))
